# baseline (speedup 1.0000x reference)
import jax
import jax.numpy as jnp
from jax.experimental import pallas as pl
from jax.experimental.pallas import tpu as pltpu

_BM = 256
_IN_FEATS = 392
_C1_COLS = 9 * 512


_DN_TB = (((1,), (1,)), ((), ()))


def _dot_t(a, b):
    return jax.lax.dot_general(a, b, _DN_TB,
                               preferred_element_type=jnp.float32)


def _net_kernel(x_ref, w1_ref, b1r_ref, w2_ref, b2r_ref,
                fw1_ref, fb1r_ref, fw2_ref, fb2_ref, out_ref):
    xb = x_ref[...].astype(jnp.bfloat16)

    p1 = _dot_t(xb, w1_ref[0:512, :])
    for t in range(1, 9):
        c = _dot_t(xb, w1_ref[t * 512:(t + 1) * 512, :])
        p1 = jnp.maximum(p1, c)
    p1 = jnp.maximum(p1 + b1r_ref[...], 0.0)

    c2 = _dot_t(p1.astype(jnp.bfloat16), w2_ref[...]) + b2r_ref[...]
    p2 = jnp.maximum(jnp.maximum(c2[:, 0:64], c2[:, 64:128]),
                     jnp.maximum(c2[:, 128:192], c2[:, 192:256]))
    p2 = jnp.maximum(p2, 0.0)

    h = _dot_t(p2.astype(jnp.bfloat16), fw1_ref[...]) + fb1r_ref[...]
    h = jnp.maximum(h, 0.0)

    o8 = _dot_t(h.astype(jnp.bfloat16), fw2_ref[...])
    o = o8[:, 0:1] + fb2_ref[...]
    out_ref[...] = jnp.tanh(jnp.maximum(o, 0.0)).astype(out_ref.dtype)


def kernel(x, w1b, b1c, w2b, b2c, fw1p, fb1c, fw2p, fb2c):
    n = x.shape[0]
    bm = _BM
    n_pad = ((n + bm - 1) // bm) * bm
    xf = x.reshape(n, _IN_FEATS)
    if n_pad != n:
        xf = jnp.pad(xf, ((0, n_pad - n), (0, 0)))

    b1r = b1c.reshape(1, 512)
    b2r = b2c.reshape(1, 256)
    fb1r = fb1c.reshape(1, 256)

    grid = (n_pad // bm,)
    in_specs = [
        pl.BlockSpec((bm, _IN_FEATS), lambda b: (b, 0)),
        pl.BlockSpec((_C1_COLS, _IN_FEATS), lambda b: (0, 0)),
        pl.BlockSpec((1, 512), lambda b: (0, 0)),
        pl.BlockSpec((256, 512), lambda b: (0, 0)),
        pl.BlockSpec((1, 256), lambda b: (0, 0)),
        pl.BlockSpec((256, 64), lambda b: (0, 0)),
        pl.BlockSpec((1, 256), lambda b: (0, 0)),
        pl.BlockSpec((8, 256), lambda b: (0, 0)),
        pl.BlockSpec((1, 1), lambda b: (0, 0)),
    ]
    out_specs = pl.BlockSpec((bm, 1), lambda b: (b, 0))

    out = pl.pallas_call(
        _net_kernel,
        out_shape=jax.ShapeDtypeStruct((n_pad, 1), jnp.float32),
        grid=grid,
        in_specs=in_specs,
        out_specs=out_specs,
        compiler_params=pltpu.CompilerParams(
            dimension_semantics=("parallel",),
            vmem_limit_bytes=64 * 1024 * 1024,
        ),
    )(xf, w1b, b1r, w2b, b2r, fw1p, fb1r, fw2p, fb2c)

    return out[:n, :].astype(x.dtype)


# speedup vs baseline: 1.0559x; 1.0559x over previous
import jax
import jax.numpy as jnp
from jax.experimental import pallas as pl
from jax.experimental.pallas import tpu as pltpu

_BM = 256
_IN_FEATS = 392
_C1_COLS = 9 * 512


_DN_TB = (((1,), (1,)), ((), ()))


def _dot_t(a, b):
    return jax.lax.dot_general(a, b, _DN_TB,
                               preferred_element_type=jnp.float32)


def _net_kernel(x_ref, w1_ref, b1r_ref, w2_ref, b2r_ref,
                fw1_ref, fb1r_ref, fw2_ref, fb2_ref, out_ref):
    xb = x_ref[...]

    p1 = _dot_t(xb, w1_ref[0:512, :])
    for t in range(1, 9):
        c = _dot_t(xb, w1_ref[t * 512:(t + 1) * 512, :])
        p1 = jnp.maximum(p1, c)
    p1 = jnp.maximum(p1 + b1r_ref[...], 0.0)

    c2 = _dot_t(p1.astype(jnp.bfloat16), w2_ref[...]) + b2r_ref[...]
    p2 = jnp.maximum(jnp.maximum(c2[:, 0:64], c2[:, 64:128]),
                     jnp.maximum(c2[:, 128:192], c2[:, 192:256]))
    p2 = jnp.maximum(p2, 0.0)

    h = _dot_t(p2.astype(jnp.bfloat16), fw1_ref[...]) + fb1r_ref[...]
    h = jnp.maximum(h, 0.0)

    o8 = _dot_t(h.astype(jnp.bfloat16), fw2_ref[...])
    o = o8[:, 0:1] + fb2_ref[...]
    out_ref[...] = jnp.tanh(jnp.maximum(o, 0.0)).astype(out_ref.dtype)


def kernel(x, w1b, b1c, w2b, b2c, fw1p, fb1c, fw2p, fb2c):
    n = x.shape[0]
    bm = _BM
    n_pad = ((n + bm - 1) // bm) * bm
    xf = x.reshape(n, _IN_FEATS).astype(jnp.bfloat16)
    if n_pad != n:
        xf = jnp.pad(xf, ((0, n_pad - n), (0, 0)))

    b1r = b1c.reshape(1, 512)
    b2r = b2c.reshape(1, 256)
    fb1r = fb1c.reshape(1, 256)

    grid = (n_pad // bm,)
    in_specs = [
        pl.BlockSpec((bm, _IN_FEATS), lambda b: (b, 0)),
        pl.BlockSpec((_C1_COLS, _IN_FEATS), lambda b: (0, 0)),
        pl.BlockSpec((1, 512), lambda b: (0, 0)),
        pl.BlockSpec((256, 512), lambda b: (0, 0)),
        pl.BlockSpec((1, 256), lambda b: (0, 0)),
        pl.BlockSpec((256, 64), lambda b: (0, 0)),
        pl.BlockSpec((1, 256), lambda b: (0, 0)),
        pl.BlockSpec((8, 256), lambda b: (0, 0)),
        pl.BlockSpec((1, 1), lambda b: (0, 0)),
    ]
    out_specs = pl.BlockSpec((bm, 1), lambda b: (b, 0))

    out = pl.pallas_call(
        _net_kernel,
        out_shape=jax.ShapeDtypeStruct((n_pad, 1), jnp.float32),
        grid=grid,
        in_specs=in_specs,
        out_specs=out_specs,
        compiler_params=pltpu.CompilerParams(
            dimension_semantics=("parallel",),
            vmem_limit_bytes=64 * 1024 * 1024,
        ),
    )(xf, w1b, b1r, w2b, b2r, fw1p, fb1r, fw2p, fb2c)

    return out[:n, :].astype(x.dtype)


# speedup vs baseline: 1.1583x; 1.0969x over previous
import jax
import jax.numpy as jnp
from jax.experimental import pallas as pl
from jax.experimental.pallas import tpu as pltpu

_BN = 256
_IN_FEATS = 392
_C1_ROWS = 9 * 512


def _net_kernel(x_ref, w1_ref, b1_ref, w2_ref, b2_ref,
                fw1_ref, fb1_ref, fw2_ref, fb2_ref, out_ref):
    f32 = jnp.float32
    xb = x_ref[...]

    p1 = jnp.dot(w1_ref[0:512, :], xb, preferred_element_type=f32)
    for t in range(1, 9):
        c = jnp.dot(w1_ref[t * 512:(t + 1) * 512, :], xb,
                    preferred_element_type=f32)
        p1 = jnp.maximum(p1, c)
    p1 = jnp.maximum(p1 + b1_ref[...], 0.0)

    c2 = jnp.dot(w2_ref[...], p1.astype(jnp.bfloat16),
                 preferred_element_type=f32) + b2_ref[...]
    p2 = jnp.maximum(jnp.maximum(c2[0:64, :], c2[64:128, :]),
                     jnp.maximum(c2[128:192, :], c2[192:256, :]))
    p2 = jnp.maximum(p2, 0.0)

    h = jnp.dot(fw1_ref[...], p2.astype(jnp.bfloat16),
                preferred_element_type=f32) + fb1_ref[...]
    h = jnp.maximum(h, 0.0)

    o8 = jnp.dot(fw2_ref[...], h.astype(jnp.bfloat16),
                 preferred_element_type=f32)
    o = o8[0:1, :] + fb2_ref[...]
    out_ref[...] = jnp.tanh(jnp.maximum(o, 0.0)).astype(out_ref.dtype)


def kernel(x, w1b, b1c, w2b, b2c, fw1p, fb1c, fw2p, fb2c):
    n = x.shape[0]
    bn = _BN
    n_pad = ((n + bn - 1) // bn) * bn

    xt = jnp.transpose(x, (2, 3, 1, 0)).reshape(_IN_FEATS, n)
    xt = xt.astype(jnp.bfloat16)
    if n_pad != n:
        xt = jnp.pad(xt, ((0, 0), (0, n_pad - n)))

    w1q = w1b.reshape(_C1_ROWS, 2, 14, 14).transpose(0, 2, 3, 1)
    w1q = w1q.reshape(_C1_ROWS, _IN_FEATS)

    grid = (n_pad // bn,)
    in_specs = [
        pl.BlockSpec((_IN_FEATS, bn), lambda b: (0, b)),
        pl.BlockSpec((_C1_ROWS, _IN_FEATS), lambda b: (0, 0)),
        pl.BlockSpec((512, 1), lambda b: (0, 0)),
        pl.BlockSpec((256, 512), lambda b: (0, 0)),
        pl.BlockSpec((256, 1), lambda b: (0, 0)),
        pl.BlockSpec((256, 64), lambda b: (0, 0)),
        pl.BlockSpec((256, 1), lambda b: (0, 0)),
        pl.BlockSpec((8, 256), lambda b: (0, 0)),
        pl.BlockSpec((1, 1), lambda b: (0, 0)),
    ]
    out_specs = pl.BlockSpec((1, bn), lambda b: (0, b))

    out_row = pl.pallas_call(
        _net_kernel,
        out_shape=jax.ShapeDtypeStruct((1, n_pad), jnp.float32),
        grid=grid,
        in_specs=in_specs,
        out_specs=out_specs,
        compiler_params=pltpu.CompilerParams(
            dimension_semantics=("parallel",),
            vmem_limit_bytes=64 * 1024 * 1024,
        ),
    )(xt, w1q, b1c, w2b, b2c, fw1p, fb1c, fw2p, fb2c)

    return out_row[0, :n].reshape(n, 1).astype(x.dtype)
